# Initial kernel scaffold; baseline (speedup 1.0000x reference)
import jax
import jax.numpy as jnp
from jax.experimental import pallas as pl


def kernel(sequence_tensor, span_indices, w_att, b_att, width_table):
    raise NotImplementedError("write your pallas kernel here")


# speedup vs baseline: 2.1737x; 2.1737x over previous
import functools

import jax
import jax.numpy as jnp
from jax.experimental import pallas as pl
from jax.experimental.pallas import tpu as pltpu

B, S, D = 8, 2048, 1024
NS = 512
NW, WD = 64, 128


def _span_body(spans_ref, seq_ref, w_ref, wt_ref, out_ref):
    seq = seq_ref[0]
    logits = jnp.dot(seq, w_ref[...],
                     preferred_element_type=jnp.float32)
    m = jnp.max(logits)
    u = jnp.exp(logits - m)
    ux = seq * u

    starts = spans_ref[0, :, 0:1]
    ends = spans_ref[0, :, 1:2]
    pos = jax.lax.broadcasted_iota(jnp.int32, (NS, S), 1)
    mask_f = ((pos >= starts) & (pos <= ends)).astype(jnp.float32)

    num = jnp.dot(mask_f, ux, preferred_element_type=jnp.float32)
    den = jnp.dot(mask_f, u, preferred_element_type=jnp.float32)
    valid = ((starts >= 0) & (ends >= starts)).astype(jnp.float32)
    emb = num * (valid / jnp.maximum(den, 1e-30))

    widths = jnp.clip(ends - starts, 0, NW - 1)
    wiota = jax.lax.broadcasted_iota(jnp.int32, (NS, NW), 1)
    onehot = (wiota == widths).astype(jnp.float32)
    wemb = jnp.dot(onehot, wt_ref[...],
                   preferred_element_type=jnp.float32)

    out_ref[0, :, :D] = emb
    out_ref[0, :, D:] = wemb


@jax.jit
def kernel(sequence_tensor, span_indices, w_att, b_att, width_table):
    del b_att
    w2 = w_att.reshape(D, 1)
    out = pl.pallas_call(
        _span_body,
        grid=(B,),
        in_specs=[
            pl.BlockSpec((1, NS, 2), lambda b: (b, 0, 0)),
            pl.BlockSpec((1, S, D), lambda b: (b, 0, 0)),
            pl.BlockSpec((D, 1), lambda b: (0, 0)),
            pl.BlockSpec((NW, WD), lambda b: (0, 0)),
        ],
        out_specs=pl.BlockSpec((1, NS, D + WD), lambda b: (b, 0, 0)),
        out_shape=jax.ShapeDtypeStruct((B, NS, D + WD), jnp.float32),
        compiler_params=pltpu.CompilerParams(
            dimension_semantics=("arbitrary",),
        ),
    )(span_indices, sequence_tensor, w2, width_table)
    return out


# speedup vs baseline: 2.2030x; 1.0135x over previous
import functools

import jax
import jax.numpy as jnp
from jax.experimental import pallas as pl
from jax.experimental.pallas import tpu as pltpu

B, S, D = 8, 2048, 1024
NS = 512
NW, WD = 64, 128


def _span_body(spans_ref, seq_ref, w_ref, wt_ref, out_ref):
    seqb = seq_ref[0].astype(jnp.bfloat16)
    logits = jnp.dot(seqb, w_ref[...].astype(jnp.bfloat16),
                     preferred_element_type=jnp.float32)
    m = jnp.max(logits)
    u = jnp.exp(logits - m)
    ub = u.astype(jnp.bfloat16)
    uxb = seqb * ub

    starts = spans_ref[0, :, 0:1]
    ends = spans_ref[0, :, 1:2]
    pos = jax.lax.broadcasted_iota(jnp.int32, (NS, S), 1)
    mask_f = ((pos >= starts) & (pos <= ends)).astype(jnp.bfloat16)

    num = jnp.dot(mask_f, uxb, preferred_element_type=jnp.float32)
    den = jnp.dot(mask_f, ub, preferred_element_type=jnp.float32)
    valid = ((starts >= 0) & (ends >= starts)).astype(jnp.float32)
    emb = num * (valid / jnp.maximum(den, 1e-30))

    widths = jnp.clip(ends - starts, 0, NW - 1)
    wiota = jax.lax.broadcasted_iota(jnp.int32, (NS, NW), 1)
    onehot = (wiota == widths).astype(jnp.float32)
    wemb = jnp.dot(onehot, wt_ref[...],
                   preferred_element_type=jnp.float32)

    out_ref[0, :, :D] = emb
    out_ref[0, :, D:] = wemb


@jax.jit
def kernel(sequence_tensor, span_indices, w_att, b_att, width_table):
    del b_att
    w2 = w_att.reshape(D, 1)
    out = pl.pallas_call(
        _span_body,
        grid=(B,),
        in_specs=[
            pl.BlockSpec((1, NS, 2), lambda b: (b, 0, 0)),
            pl.BlockSpec((1, S, D), lambda b: (b, 0, 0)),
            pl.BlockSpec((D, 1), lambda b: (0, 0)),
            pl.BlockSpec((NW, WD), lambda b: (0, 0)),
        ],
        out_specs=pl.BlockSpec((1, NS, D + WD), lambda b: (b, 0, 0)),
        out_shape=jax.ShapeDtypeStruct((B, NS, D + WD), jnp.float32),
        compiler_params=pltpu.CompilerParams(
            dimension_semantics=("arbitrary",),
        ),
    )(span_indices, sequence_tensor, w2, width_table)
    return out
